# Initial kernel scaffold; baseline (speedup 1.0000x reference)
import jax
import jax.numpy as jnp
from jax.experimental import pallas as pl


def kernel(x, edge_index, W1a, b1a, W1b, b1b, W2a, b2a, W2b, b2b, Wf1, bf1, gamma, beta, Wf2, bf2):
    raise NotImplementedError("write your pallas kernel here")


# speedup vs baseline: 7.1160x; 7.1160x over previous
import functools

import jax
import jax.numpy as jnp
from jax import lax
from jax.experimental import pallas as pl
from jax.experimental.pallas import tpu as pltpu
from jax.experimental.pallas import tpu_sc as plsc

N = 39040
F = 128
E = 624640
NGRAPH = N // F

NC = 2
NS = 16
L = 16

NR = 4
RSZ = N // NR
PASSES = NR // NC
PAD_ROWS = 2 * L
EPT = E // NS
CH = 1952
NCH = EPT // CH
G = 128
CCH = 80
NCPY = RSZ // CCH
SEL = 4096


def _agg_body(h, src, dst, out, acc, src_chunk, dst_chunk, src_sel, dst_sel,
              src_stage, dst_stage, rows, sem):
    c = lax.axis_index("c")
    s = lax.axis_index("s")
    wid = s * NC + c

    for p in range(PASSES):
        r = c * PASSES + p
        lo = r * RSZ

        for k in range((NCPY + NS - 1) // NS):
            ci = s + k * NS

            @pl.when(ci < NCPY)
            def _(ci=ci):
                rb = pl.multiple_of(ci * CCH, 8)
                pltpu.sync_copy(h.at[pl.ds(pl.multiple_of(lo + rb, 8), CCH)],
                                rows.at[pl.ds(0, CCH)])
                pltpu.sync_copy(rows.at[pl.ds(0, CCH)],
                                acc.at[pl.ds(rb, CCH)])

        plsc.subcore_barrier()

        ebase = s * EPT

        def drain(off, cons):
            ngc = (off - cons) // G

            def group_body(g, cons):
                gb = cons & (SEL - 1)
                for k in range(G // L):
                    src_stage[pl.ds(k * L, L)] = src_sel[pl.ds(gb + k * L, L)]
                    dst_stage[pl.ds(k * L, L)] = dst_sel[pl.ds(gb + k * L, L)]
                pltpu.async_copy(h.at[src_stage], rows, sem).wait()
                pltpu.sync_copy(rows, acc.at[dst_stage], add=True)
                return cons + G

            return lax.fori_loop(0, ngc, group_body, cons)

        def chunk_body(ci, carry):
            off, cons = carry
            eb = pl.multiple_of(ebase + ci * CH, 8)
            pltpu.sync_copy(src.at[pl.ds(eb, CH)], src_chunk)
            pltpu.sync_copy(dst.at[pl.ds(eb, CH)], dst_chunk)

            def vec_body(j, off):
                vs = src_chunk[pl.ds(j * L, L)]
                vd = dst_chunk[pl.ds(j * L, L)]
                m = (vd >= lo) & (vd < lo + RSZ)
                mi = m.astype(jnp.int32)
                pos = (off + plsc.cumsum(mi) - 1) & (SEL - 1)
                plsc.store_scatter(src_sel, [pos], vs, mask=m)
                plsc.store_scatter(dst_sel, [pos], vd - lo, mask=m)
                return off + jnp.sum(mi)

            off = lax.fori_loop(0, CH // L, vec_body, off)
            return off, drain(off, cons)

        off, cons = lax.fori_loop(0, NCH, chunk_body,
                                  (jnp.int32(0), jnp.int32(0)))

        pad_s = wid * L + lax.iota(jnp.int32, L)
        pad_d = RSZ + lax.iota(jnp.int32, L)

        @pl.when(off > cons)
        def _():
            for k in range(G // L):
                pos = (off + k * L + lax.iota(jnp.int32, L)) & (SEL - 1)
                plsc.store_scatter(src_sel, [pos], pad_s)
                plsc.store_scatter(dst_sel, [pos], pad_d + (k % 2) * L)

        drain(((off + G - 1) // G) * G, cons)
        plsc.subcore_barrier()

        for k in range((NCPY + NS - 1) // NS):
            ci = s + k * NS

            @pl.when(ci < NCPY)
            def _(ci=ci):
                rb = pl.multiple_of(ci * CCH, 8)
                pltpu.sync_copy(acc.at[pl.ds(rb, CCH)], rows.at[pl.ds(0, CCH)])
                pltpu.sync_copy(rows.at[pl.ds(0, CCH)],
                                out.at[pl.ds(pl.multiple_of(lo + rb, 8), CCH)])

        plsc.subcore_barrier()


_aggregate = functools.partial(
    pl.kernel,
    out_type=jax.ShapeDtypeStruct((N, F), jnp.float32),
    mesh=plsc.VectorSubcoreMesh(core_axis_name="c", subcore_axis_name="s",
                                num_cores=NC, num_subcores=NS),
    scratch_types=[
        pltpu.VMEM_SHARED((RSZ + PAD_ROWS, F), jnp.float32),
        pltpu.VMEM((CH,), jnp.int32),
        pltpu.VMEM((CH,), jnp.int32),
        pltpu.VMEM((SEL,), jnp.int32),
        pltpu.VMEM((SEL,), jnp.int32),
        pltpu.VMEM((G,), jnp.int32),
        pltpu.VMEM((G,), jnp.int32),
        pltpu.VMEM((G, F), jnp.float32),
        pltpu.SemaphoreType.DMA,
    ],
    compiler_params=pltpu.CompilerParams(needs_layout_passes=False),
)(_agg_body)


BLK = 2440


def _conv_block(z_ref, wa_ref, ba_ref, wb_ref, bb_ref, o_ref):
    z = z_ref[...]
    t = jnp.maximum(
        jnp.dot(z, wa_ref[...], preferred_element_type=jnp.float32)
        + ba_ref[...], 0.0)
    o_ref[...] = jnp.maximum(
        jnp.dot(t, wb_ref[...], preferred_element_type=jnp.float32)
        + bb_ref[...], 0.0)


def _conv(z, wa, ba, wb, bb):
    return pl.pallas_call(
        _conv_block,
        grid=(N // BLK,),
        in_specs=[
            pl.BlockSpec((BLK, F), lambda i: (i, 0)),
            pl.BlockSpec((F, F), lambda i: (0, 0)),
            pl.BlockSpec((1, F), lambda i: (0, 0)),
            pl.BlockSpec((F, F), lambda i: (0, 0)),
            pl.BlockSpec((1, F), lambda i: (0, 0)),
        ],
        out_specs=pl.BlockSpec((BLK, F), lambda i: (i, 0)),
        out_shape=jax.ShapeDtypeStruct((N, F), jnp.float32),
    )(z, wa, ba.reshape(1, F), wb, bb.reshape(1, F))


KCH = 2048
BN_SCALE = 1.0 / (1.0 + 1e-5) ** 0.5


def _head_block(hf_ref, w1_ref, bf1_ref, gamma_ref, beta_ref, w2_ref, bf2_ref,
                o_ref, acc_ref):
    k = pl.program_id(0)

    @pl.when(k == 0)
    def _():
        acc_ref[...] = jnp.zeros_like(acc_ref)

    acc_ref[...] += jnp.dot(hf_ref[...], w1_ref[...],
                            preferred_element_type=jnp.float32)

    @pl.when(k == pl.num_programs(0) - 1)
    def _():
        o = acc_ref[...] + bf1_ref[...]
        o = o * (BN_SCALE * gamma_ref[...]) + beta_ref[...]
        o = jnp.maximum(o, 0.0)
        o_ref[...] = (jnp.dot(o, w2_ref[...],
                              preferred_element_type=jnp.float32)
                      + bf2_ref[...])


def _head(hf, w1, bf1, gamma, beta, w2, bf2):
    kd = F * F
    return pl.pallas_call(
        _head_block,
        grid=(kd // KCH,),
        in_specs=[
            pl.BlockSpec((NGRAPH, KCH), lambda k: (0, k)),
            pl.BlockSpec((KCH, F), lambda k: (k, 0)),
            pl.BlockSpec((1, F), lambda k: (0, 0)),
            pl.BlockSpec((1, F), lambda k: (0, 0)),
            pl.BlockSpec((1, F), lambda k: (0, 0)),
            pl.BlockSpec((F, 2), lambda k: (0, 0)),
            pl.BlockSpec((1, 2), lambda k: (0, 0)),
        ],
        out_specs=pl.BlockSpec((NGRAPH, 2), lambda k: (0, 0)),
        out_shape=jax.ShapeDtypeStruct((NGRAPH, 2), jnp.float32),
        scratch_shapes=[pltpu.VMEM((NGRAPH, F), jnp.float32)],
    )(hf, w1, bf1.reshape(1, F), gamma.reshape(1, F), beta.reshape(1, F),
      w2, bf2.reshape(1, 2))


def kernel(x, edge_index, W1a, b1a, W1b, b1b, W2a, b2a, W2b, b2b,
           Wf1, bf1, gamma, beta, Wf2, bf2):
    src = edge_index[0]
    dst = edge_index[1]
    z1 = _aggregate(x, src, dst)
    h1 = _conv(z1, W1a, b1a, W1b, b1b)
    z2 = _aggregate(h1, src, dst)
    h2 = _conv(z2, W2a, b2a, W2b, b2b)
    hf = h2.reshape(NGRAPH, F * F)
    return _head(hf, Wf1, bf1, gamma, beta, Wf2, bf2)
